# baseline (speedup 1.0000x reference)
import functools

import jax
import jax.numpy as jnp
from jax import lax
from jax.experimental import pallas as pl
from jax.experimental.pallas import tpu as pltpu
from jax.experimental.pallas import tpu_sc as plsc


_ROW_BLK = 400
_PAIR_BLK = 2048


def _xw_body(h_ref, w_ref, o_ref):
    o_ref[...] = jnp.dot(h_ref[...].astype(jnp.bfloat16), w_ref[...],
                         preferred_element_type=jnp.float32).astype(jnp.bfloat16)


def _layer1_body(a_ref, x0_ref, w_ref, x1_ref, a8_ref):
    a = a_ref[...]
    acc = jnp.dot(a.astype(jnp.bfloat16), x0_ref[...],
                  preferred_element_type=jnp.float32)
    h = jnp.maximum(acc, 0.0).astype(jnp.bfloat16)
    x1_ref[...] = jnp.dot(h, w_ref[...],
                          preferred_element_type=jnp.float32).astype(jnp.float8_e4m3fn)
    a8_ref[...] = a.astype(jnp.float8_e4m3fn)


def _layer2_body(a8_ref, x_ref, o_ref):
    acc = jnp.dot(a8_ref[...], x_ref[...],
                  preferred_element_type=jnp.float32)
    o_ref[...] = jnp.maximum(acc, 0.0)


def _mlp_body(se_ref, de_ref, lab_ref, w1_ref, b1_ref, w2_ref, b2_ref, o_ref,
              *, d_model, b_total):
    i = pl.program_id(0)
    w1 = w1_ref[...].astype(jnp.bfloat16)
    h = jnp.dot(se_ref[...].astype(jnp.bfloat16), w1[:d_model],
                preferred_element_type=jnp.float32)
    h += jnp.dot(de_ref[...].astype(jnp.bfloat16), w1[d_model:],
                 preferred_element_type=jnp.float32)
    h = jnp.maximum(h + b1_ref[...], 0.0)
    l = jnp.dot(h, w2_ref[...], preferred_element_type=jnp.float32)
    b2 = b2_ref[...]
    d = (l[:, 1:2] - l[:, 0:1]) + (b2[:, 1:2] - b2[:, 0:1])
    z = jnp.reshape(d * 0.5, (d.shape[0] // 128, 128))
    u = jnp.tanh(z)
    s = 1.0 - 2.0 * lab_ref[...].astype(jnp.float32)
    nll = jnp.log1p(jnp.exp(s * u))
    part = jnp.sum(nll) * (1.0 / b_total)

    @pl.when(i == 0)
    def _():
        o_ref[...] = jnp.zeros_like(o_ref)

    o_ref[...] += part


_SC_WORKERS = 32
_SC_CHUNK = 128


def _make_sc_gather(n_idx, d_model, dtype):
    per_w = n_idx // _SC_WORKERS
    n_chunks = per_w // _SC_CHUNK
    mesh = plsc.VectorSubcoreMesh(core_axis_name="c", subcore_axis_name="s")

    @functools.partial(
        pl.kernel, mesh=mesh,
        out_type=jax.ShapeDtypeStruct((n_idx, d_model), dtype),
        scratch_types=[
            pltpu.VMEM((n_chunks, _SC_CHUNK), jnp.int32),
            pltpu.VMEM((2, _SC_CHUNK, d_model), dtype),
            pltpu.SemaphoreType.DMA,
            pltpu.SemaphoreType.DMA,
            pltpu.SemaphoreType.DMA,
            pltpu.SemaphoreType.DMA,
        ],
    )
    def _gather(table_hbm, idx_hbm, out_hbm, idx_v, rows_v, g0, g1, o0, o1):
        wid = lax.axis_index("s") * 2 + lax.axis_index("c")
        base = wid * n_chunks
        pltpu.sync_copy(idx_hbm.at[pl.ds(base, n_chunks)], idx_v)
        gsem = (g0, g1)
        osem = (o0, o1)
        gathers = [None, None]
        outs = [None, None]
        gathers[0] = pltpu.async_copy(table_hbm.at[idx_v.at[0]],
                                      rows_v.at[0], gsem[0])
        for c in range(n_chunks):
            b = c % 2
            nb = (c + 1) % 2
            if c + 1 < n_chunks:
                if outs[nb] is not None:
                    outs[nb].wait()
                gathers[nb] = pltpu.async_copy(table_hbm.at[idx_v.at[c + 1]],
                                               rows_v.at[nb], gsem[nb])
            gathers[b].wait()
            outs[b] = pltpu.async_copy(
                rows_v.at[b],
                out_hbm.at[pl.ds((base + c) * _SC_CHUNK, _SC_CHUNK)],
                osem[b])
        for b in range(2):
            if outs[b] is not None:
                outs[b].wait()

    return _gather


def kernel(pairs, labels, A, emb, Ws, W1, b1, W2, b2):
    n, d = emb.shape
    b_pairs = pairs.shape[0]
    wsb = Ws.astype(jnp.bfloat16)

    x0 = pl.pallas_call(
        _xw_body,
        out_shape=jax.ShapeDtypeStruct((n, d), jnp.bfloat16),
    )(emb, wsb[0])

    x1, a8 = pl.pallas_call(
        _layer1_body,
        grid=(n // _ROW_BLK,),
        in_specs=[
            pl.BlockSpec((_ROW_BLK, n), lambda i: (i, 0)),
            pl.BlockSpec((n, d), lambda i: (0, 0)),
            pl.BlockSpec((d, d), lambda i: (0, 0)),
        ],
        out_specs=[pl.BlockSpec((_ROW_BLK, d), lambda i: (i, 0)),
                   pl.BlockSpec((_ROW_BLK, n), lambda i: (i, 0))],
        out_shape=[jax.ShapeDtypeStruct((n, d), jnp.float8_e4m3fn),
                   jax.ShapeDtypeStruct((n, n), jnp.float8_e4m3fn)],
        compiler_params=pltpu.CompilerParams(
            vmem_limit_bytes=60 * 1024 * 1024,
            dimension_semantics=("parallel",),
        ),
    )(A, x0, wsb[1])

    h2 = pl.pallas_call(
        _layer2_body,
        grid=(n // _ROW_BLK,),
        in_specs=[
            pl.BlockSpec((_ROW_BLK, n), lambda i: (i, 0)),
            pl.BlockSpec((n, d), lambda i: (0, 0)),
        ],
        out_specs=pl.BlockSpec((_ROW_BLK, d), lambda i: (i, 0)),
        out_shape=jax.ShapeDtypeStruct((n, d), jnp.float32),
        compiler_params=pltpu.CompilerParams(
            vmem_limit_bytes=60 * 1024 * 1024,
            dimension_semantics=("parallel",),
        ),
    )(a8, x1)

    idx = jnp.concatenate([pairs[:, 0], pairs[:, 1]]).astype(jnp.int32)
    gathered = _make_sc_gather(2 * b_pairs, d, jnp.float32)(
        h2, idx.reshape(-1, _SC_CHUNK))

    n_blk = b_pairs // _PAIR_BLK
    loss = pl.pallas_call(
        functools.partial(_mlp_body, d_model=d, b_total=float(b_pairs)),
        grid=(n_blk,),
        in_specs=[
            pl.BlockSpec((_PAIR_BLK, d), lambda i: (i, 0)),
            pl.BlockSpec((_PAIR_BLK, d), lambda i: (n_blk + i, 0)),
            pl.BlockSpec((_PAIR_BLK // 128, 128), lambda i: (i, 0)),
            pl.BlockSpec((2 * d, d), lambda i: (0, 0)),
            pl.BlockSpec((1, d), lambda i: (0, 0)),
            pl.BlockSpec((d, 2), lambda i: (0, 0)),
            pl.BlockSpec((1, 2), lambda i: (0, 0)),
        ],
        out_specs=pl.BlockSpec((1, 1), lambda i: (0, 0)),
        out_shape=jax.ShapeDtypeStruct((1, 1), jnp.float32),
    )(gathered, gathered, labels.astype(jnp.int32).reshape(b_pairs // 128, 128),
      W1, b1.reshape(1, d), W2, b2.reshape(1, 2))

    return loss.reshape(())


# speedup vs baseline: 1.0169x; 1.0169x over previous
import functools

import jax
import jax.numpy as jnp
from jax import lax
from jax.experimental import pallas as pl
from jax.experimental.pallas import tpu as pltpu
from jax.experimental.pallas import tpu_sc as plsc


_ROW_BLK = 400
_PAIR_BLK = 2048


def _layer1_body(a_ref, emb_ref, w_ref, x1_ref, a8_ref, x0_scr):
    i = pl.program_id(0)

    @pl.when(i == 0)
    def _():
        x0_scr[...] = jnp.dot(emb_ref[...].astype(jnp.bfloat16), w_ref[0],
                              preferred_element_type=jnp.float32).astype(jnp.bfloat16)

    a = a_ref[...]
    acc = jnp.dot(a.astype(jnp.bfloat16), x0_scr[...],
                  preferred_element_type=jnp.float32)
    h = jnp.maximum(acc, 0.0).astype(jnp.bfloat16)
    x1_ref[...] = jnp.dot(h, w_ref[1],
                          preferred_element_type=jnp.float32).astype(jnp.float8_e4m3fn)
    a8_ref[...] = a.astype(jnp.float8_e4m3fn)


def _layer2_body(a8_ref, x_ref, o_ref):
    acc = jnp.dot(a8_ref[...], x_ref[...],
                  preferred_element_type=jnp.float32)
    o_ref[...] = jnp.maximum(acc, 0.0)


def _mlp_body(se_ref, de_ref, lab_ref, w1_ref, b1_ref, w2_ref, b2_ref, o_ref,
              *, d_model, b_total):
    i = pl.program_id(0)
    w1 = w1_ref[...].astype(jnp.bfloat16)
    h = jnp.dot(se_ref[...].astype(jnp.bfloat16), w1[:d_model],
                preferred_element_type=jnp.float32)
    h += jnp.dot(de_ref[...].astype(jnp.bfloat16), w1[d_model:],
                 preferred_element_type=jnp.float32)
    h = jnp.maximum(h + b1_ref[...], 0.0)
    l = jnp.dot(h, w2_ref[...], preferred_element_type=jnp.float32)
    b2 = b2_ref[...]
    d = (l[:, 1:2] - l[:, 0:1]) + (b2[:, 1:2] - b2[:, 0:1])
    z = jnp.reshape(d * 0.5, (d.shape[0] // 128, 128))
    u = jnp.tanh(z)
    s = 1.0 - 2.0 * lab_ref[...].astype(jnp.float32)
    nll = jnp.log1p(jnp.exp(s * u))
    part = jnp.sum(nll) * (1.0 / b_total)

    @pl.when(i == 0)
    def _():
        o_ref[...] = jnp.zeros_like(o_ref)

    o_ref[...] += part


_SC_WORKERS = 32
_SC_CHUNK = 128


def _make_sc_gather(n_idx, d_model, dtype):
    per_w = n_idx // _SC_WORKERS
    n_chunks = per_w // _SC_CHUNK
    mesh = plsc.VectorSubcoreMesh(core_axis_name="c", subcore_axis_name="s")

    @functools.partial(
        pl.kernel, mesh=mesh,
        out_type=jax.ShapeDtypeStruct((n_idx, d_model), dtype),
        scratch_types=[
            pltpu.VMEM((n_chunks, _SC_CHUNK), jnp.int32),
            pltpu.VMEM((4, _SC_CHUNK, d_model), dtype),
        ] + [pltpu.SemaphoreType.DMA] * 8,
    )
    def _gather(table_hbm, idx_hbm, out_hbm, idx_v, rows_v, *sems):
        wid = lax.axis_index("s") * 2 + lax.axis_index("c")
        base = wid * n_chunks
        pltpu.sync_copy(idx_hbm.at[pl.ds(base, n_chunks)], idx_v)
        gsem = sems[:4]
        osem = sems[4:]
        gathers = [None] * 4
        outs = [None] * 4
        depth = min(3, n_chunks)
        for c in range(depth):
            gathers[c] = pltpu.async_copy(table_hbm.at[idx_v.at[c]],
                                          rows_v.at[c], gsem[c])
        for c in range(n_chunks):
            b = c % 4
            p = c + depth
            if p < n_chunks:
                pb = p % 4
                if outs[pb] is not None:
                    outs[pb].wait()
                    outs[pb] = None
                gathers[pb] = pltpu.async_copy(table_hbm.at[idx_v.at[p]],
                                               rows_v.at[pb], gsem[pb])
            gathers[b].wait()
            outs[b] = pltpu.async_copy(
                rows_v.at[b],
                out_hbm.at[pl.ds((base + c) * _SC_CHUNK, _SC_CHUNK)],
                osem[b])
        for b in range(4):
            if outs[b] is not None:
                outs[b].wait()

    return _gather


def kernel(pairs, labels, A, emb, Ws, W1, b1, W2, b2):
    n, d = emb.shape
    b_pairs = pairs.shape[0]
    wsb = Ws.astype(jnp.bfloat16)

    x1, a8 = pl.pallas_call(
        _layer1_body,
        grid=(n // _ROW_BLK,),
        in_specs=[
            pl.BlockSpec((_ROW_BLK, n), lambda i: (i, 0)),
            pl.BlockSpec((n, d), lambda i: (0, 0)),
            pl.BlockSpec((2, d, d), lambda i: (0, 0, 0)),
        ],
        out_specs=[pl.BlockSpec((_ROW_BLK, d), lambda i: (i, 0)),
                   pl.BlockSpec((_ROW_BLK, n), lambda i: (i, 0))],
        out_shape=[jax.ShapeDtypeStruct((n, d), jnp.float8_e4m3fn),
                   jax.ShapeDtypeStruct((n, n), jnp.float8_e4m3fn)],
        scratch_shapes=[pltpu.VMEM((n, d), jnp.bfloat16)],
        compiler_params=pltpu.CompilerParams(
            vmem_limit_bytes=60 * 1024 * 1024,
        ),
    )(A, emb, wsb)

    h2 = pl.pallas_call(
        _layer2_body,
        grid=(n // _ROW_BLK,),
        in_specs=[
            pl.BlockSpec((_ROW_BLK, n), lambda i: (i, 0)),
            pl.BlockSpec((n, d), lambda i: (0, 0)),
        ],
        out_specs=pl.BlockSpec((_ROW_BLK, d), lambda i: (i, 0)),
        out_shape=jax.ShapeDtypeStruct((n, d), jnp.float32),
        compiler_params=pltpu.CompilerParams(
            vmem_limit_bytes=60 * 1024 * 1024,
        ),
    )(a8, x1)

    idx = jnp.concatenate([pairs[:, 0], pairs[:, 1]]).astype(jnp.int32)
    gathered = _make_sc_gather(2 * b_pairs, d, jnp.float32)(
        h2, idx.reshape(-1, _SC_CHUNK))

    n_blk = b_pairs // _PAIR_BLK
    loss = pl.pallas_call(
        functools.partial(_mlp_body, d_model=d, b_total=float(b_pairs)),
        grid=(n_blk,),
        in_specs=[
            pl.BlockSpec((_PAIR_BLK, d), lambda i: (i, 0)),
            pl.BlockSpec((_PAIR_BLK, d), lambda i: (n_blk + i, 0)),
            pl.BlockSpec((_PAIR_BLK // 128, 128), lambda i: (i, 0)),
            pl.BlockSpec((2 * d, d), lambda i: (0, 0)),
            pl.BlockSpec((1, d), lambda i: (0, 0)),
            pl.BlockSpec((d, 2), lambda i: (0, 0)),
            pl.BlockSpec((1, 2), lambda i: (0, 0)),
        ],
        out_specs=pl.BlockSpec((1, 1), lambda i: (0, 0)),
        out_shape=jax.ShapeDtypeStruct((1, 1), jnp.float32),
    )(gathered, gathered, labels.astype(jnp.int32).reshape(b_pairs // 128, 128),
      W1, b1.reshape(1, d), W2, b2.reshape(1, 2))

    return loss.reshape(())


# speedup vs baseline: 1.0325x; 1.0153x over previous
import functools

import jax
import jax.numpy as jnp
from jax import lax
from jax.experimental import pallas as pl
from jax.experimental.pallas import tpu as pltpu
from jax.experimental.pallas import tpu_sc as plsc


_ROW_BLK = 400
_PAIR_BLK = 2048


def _layer1_body(a_ref, emb_ref, w_ref, x1_ref, a8_ref, x0_scr):
    i = pl.program_id(0)

    @pl.when(i == 0)
    def _():
        x0_scr[...] = jnp.dot(emb_ref[...].astype(jnp.bfloat16), w_ref[0],
                              preferred_element_type=jnp.float32).astype(jnp.bfloat16)

    a = a_ref[...]
    acc = jnp.dot(a.astype(jnp.bfloat16), x0_scr[...],
                  preferred_element_type=jnp.float32)
    h = jnp.maximum(acc, 0.0).astype(jnp.bfloat16)
    x1_ref[...] = jnp.dot(h, w_ref[1],
                          preferred_element_type=jnp.float32).astype(jnp.float8_e4m3fn)
    a8_ref[...] = a.astype(jnp.float8_e4m3fn)


def _layer2_body(a8_ref, x_ref, o_ref):
    acc = jnp.dot(a8_ref[...], x_ref[...],
                  preferred_element_type=jnp.float32)
    o_ref[...] = jnp.maximum(acc, 0.0)


def _mlp_body(se_ref, de_ref, lab_ref, w1_ref, b1_ref, w2_ref, b2_ref, o_ref,
              *, d_model, b_total):
    i = pl.program_id(0)
    w1 = w1_ref[...].astype(jnp.bfloat16)
    h = jnp.dot(se_ref[...].astype(jnp.bfloat16), w1[:d_model],
                preferred_element_type=jnp.float32)
    h += jnp.dot(de_ref[...].astype(jnp.bfloat16), w1[d_model:],
                 preferred_element_type=jnp.float32)
    h = jnp.maximum(h + b1_ref[...], 0.0)
    l = jnp.dot(h, w2_ref[...], preferred_element_type=jnp.float32)
    b2 = b2_ref[...]
    d = (l[:, 1:2] - l[:, 0:1]) + (b2[:, 1:2] - b2[:, 0:1])
    z = jnp.reshape(d * 0.5, (d.shape[0] // 128, 128))
    u = jnp.tanh(z)
    s = 1.0 - 2.0 * lab_ref[...].astype(jnp.float32)
    nll = jnp.log1p(jnp.exp(s * u))
    part = jnp.sum(nll) * (1.0 / b_total)

    @pl.when(i == 0)
    def _():
        o_ref[...] = jnp.zeros_like(o_ref)

    o_ref[...] += part


_SC_WORKERS = 32
_SC_CHUNK = 128


def _make_sc_gather(n_idx, d_model, dtype):
    per_w = n_idx // _SC_WORKERS
    n_chunks = per_w // _SC_CHUNK
    mesh = plsc.VectorSubcoreMesh(core_axis_name="c", subcore_axis_name="s")

    @functools.partial(
        pl.kernel, mesh=mesh,
        out_type=jax.ShapeDtypeStruct((n_idx, d_model), dtype),
        scratch_types=[
            pltpu.VMEM((n_chunks, _SC_CHUNK), jnp.int32),
            pltpu.VMEM((4, _SC_CHUNK, d_model), dtype),
        ] + [pltpu.SemaphoreType.DMA] * 8,
    )
    def _gather(table_hbm, idx_hbm, out_hbm, idx_v, rows_v, *sems):
        wid = lax.axis_index("s") * 2 + lax.axis_index("c")
        base = wid * n_chunks
        pltpu.sync_copy(idx_hbm.at[pl.ds(base, n_chunks)], idx_v)
        gsem = sems[:4]
        osem = sems[4:]
        gathers = [None] * 4
        outs = [None] * 4
        depth = min(3, n_chunks)
        for c in range(depth):
            gathers[c] = pltpu.async_copy(table_hbm.at[idx_v.at[c]],
                                          rows_v.at[c], gsem[c])
        for c in range(n_chunks):
            b = c % 4
            p = c + depth
            if p < n_chunks:
                pb = p % 4
                if outs[pb] is not None:
                    outs[pb].wait()
                    outs[pb] = None
                gathers[pb] = pltpu.async_copy(table_hbm.at[idx_v.at[p]],
                                               rows_v.at[pb], gsem[pb])
            gathers[b].wait()
            outs[b] = pltpu.async_copy(
                rows_v.at[b],
                out_hbm.at[pl.ds((base + c) * _SC_CHUNK, _SC_CHUNK)],
                osem[b])
        for b in range(4):
            if outs[b] is not None:
                outs[b].wait()

    return _gather


def kernel(pairs, labels, A, emb, Ws, W1, b1, W2, b2):
    n, d = emb.shape
    b_pairs = pairs.shape[0]
    wsb = Ws.astype(jnp.bfloat16)

    x1, a8 = pl.pallas_call(
        _layer1_body,
        grid=(n // _ROW_BLK,),
        in_specs=[
            pl.BlockSpec((_ROW_BLK, n), lambda i: (i, 0)),
            pl.BlockSpec((n, d), lambda i: (0, 0)),
            pl.BlockSpec((2, d, d), lambda i: (0, 0, 0)),
        ],
        out_specs=[pl.BlockSpec((_ROW_BLK, d), lambda i: (i, 0)),
                   pl.BlockSpec((_ROW_BLK, n), lambda i: (i, 0))],
        out_shape=[jax.ShapeDtypeStruct((n, d), jnp.float8_e4m3fn),
                   jax.ShapeDtypeStruct((n, n), jnp.float8_e4m3fn)],
        scratch_shapes=[pltpu.VMEM((n, d), jnp.bfloat16)],
        compiler_params=pltpu.CompilerParams(
            vmem_limit_bytes=60 * 1024 * 1024,
        ),
    )(A, emb, wsb)

    h2 = pl.pallas_call(
        _layer2_body,
        grid=(n // (_ROW_BLK * 5),),
        in_specs=[
            pl.BlockSpec((_ROW_BLK * 5, n), lambda i: (i, 0)),
            pl.BlockSpec((n, d), lambda i: (0, 0)),
        ],
        out_specs=pl.BlockSpec((_ROW_BLK * 5, d), lambda i: (i, 0)),
        out_shape=jax.ShapeDtypeStruct((n, d), jnp.float32),
        compiler_params=pltpu.CompilerParams(
            vmem_limit_bytes=60 * 1024 * 1024,
        ),
    )(a8, x1)

    idx = jnp.concatenate([pairs[:, 0], pairs[:, 1]]).astype(jnp.int32)
    gathered = _make_sc_gather(2 * b_pairs, d, jnp.float32)(
        h2, idx.reshape(-1, _SC_CHUNK))

    n_blk = b_pairs // _PAIR_BLK
    loss = pl.pallas_call(
        functools.partial(_mlp_body, d_model=d, b_total=float(b_pairs)),
        grid=(n_blk,),
        in_specs=[
            pl.BlockSpec((_PAIR_BLK, d), lambda i: (i, 0)),
            pl.BlockSpec((_PAIR_BLK, d), lambda i: (n_blk + i, 0)),
            pl.BlockSpec((_PAIR_BLK // 128, 128), lambda i: (i, 0)),
            pl.BlockSpec((2 * d, d), lambda i: (0, 0)),
            pl.BlockSpec((1, d), lambda i: (0, 0)),
            pl.BlockSpec((d, 2), lambda i: (0, 0)),
            pl.BlockSpec((1, 2), lambda i: (0, 0)),
        ],
        out_specs=pl.BlockSpec((1, 1), lambda i: (0, 0)),
        out_shape=jax.ShapeDtypeStruct((1, 1), jnp.float32),
    )(gathered, gathered, labels.astype(jnp.int32).reshape(b_pairs // 128, 128),
      W1, b1.reshape(1, d), W2, b2.reshape(1, 2))

    return loss.reshape(())
